# Initial kernel scaffold; baseline (speedup 1.0000x reference)
import jax
import jax.numpy as jnp
from jax.experimental import pallas as pl


def kernel(seq_len, relative_bias):
    raise NotImplementedError("write your pallas kernel here")


# speedup vs baseline: 42.5252x; 42.5252x over previous
import functools

import jax
import jax.numpy as jnp
from jax import lax
from jax.experimental import pallas as pl
from jax.experimental.pallas import tpu as pltpu
from jax.experimental.pallas import tpu_sc as plsc

_H = 16
_S = 2048
_MAXD = 128
_DLEN = 4224
_TPAD = 264
_LEAD = 4
_GROUPS = _S // 2 // 8


def _bias_body(table_hbm, out_hbm, tab_v, d8_v, sem):
    core = lax.axis_index("c")
    sub = lax.axis_index("s")
    wid = sub * 2 + core
    h = wid // 2
    half = wid % 2
    i_base = half * (_S // 2)

    pltpu.sync_copy(table_hbm, tab_v)

    zeros = jnp.zeros((16,), jnp.int32)
    hvec = zeros + h
    t0 = plsc.load_gather(tab_v, [hvec, zeros])
    t256 = plsc.load_gather(tab_v, [hvec, zeros + 2 * _MAXD])

    def fill_left(c, _):
        d8_v[7, pl.ds(c * 16, 16)] = t0
        return _

    lax.fori_loop(0, 120, fill_left, None)

    def fill_mid(c, _):
        d8_v[7, pl.ds(1920 + c * 16, 16)] = tab_v[h, pl.ds(1 + c * 16, 16)]
        return _

    lax.fori_loop(0, 16, fill_mid, None)

    def fill_right(c, _):
        d8_v[7, pl.ds(2176 + c * 16, 16)] = t256
        return _

    lax.fori_loop(0, 128, fill_right, None)

    for t in range(7):
        s = 7 - t

        def shift(c, _, s=s, t=t):
            d8_v[t, pl.ds(c * 16, 16)] = d8_v[7, pl.ds(c * 16 + s, 16)]
            return _

        lax.fori_loop(0, 256, shift, None)

    def group(g, _):
        i0 = i_base + g * 8
        start8 = 2040 - i0
        cp = pltpu.make_async_copy(
            d8_v.at[:, pl.ds(start8, _S)],
            out_hbm.at[h, pl.ds(i0, 8)],
            sem,
        )
        cp.start()

        @pl.when(g >= _LEAD)
        def _drain():
            pltpu.make_async_copy(
                d8_v.at[:, pl.ds(0, _S)],
                out_hbm.at[h, pl.ds(i_base, 8)],
                sem,
            ).wait()

        return _

    lax.fori_loop(0, _GROUPS, group, None)
    for _ in range(_LEAD):
        pltpu.make_async_copy(
            d8_v.at[:, pl.ds(0, _S)],
            out_hbm.at[0, pl.ds(0, 8)],
            sem,
        ).wait()


@functools.partial(jax.jit, static_argnums=())
def _bias_call(tableT):
    mesh = plsc.VectorSubcoreMesh(core_axis_name="c", subcore_axis_name="s")
    return pl.kernel(
        _bias_body,
        out_type=jax.ShapeDtypeStruct((_H, _S, _S), jnp.float32),
        mesh=mesh,
        scratch_types=[
            pltpu.VMEM((_H, _TPAD), jnp.float32),
            pltpu.VMEM((8, _DLEN), jnp.float32),
            pltpu.SemaphoreType.DMA,
        ],
        compiler_params=pltpu.CompilerParams(
            use_tc_tiling_on_sc=False, needs_layout_passes=False
        ),
    )(tableT)


def kernel(seq_len, relative_bias):
    del seq_len
    tableT = jnp.transpose(relative_bias)
    tableT = jnp.pad(tableT, ((0, 0), (0, _TPAD - 2 * _MAXD - 1)))
    return _bias_call(tableT)


# speedup vs baseline: 142.6501x; 3.3545x over previous
import functools

import jax
import jax.numpy as jnp
from jax import lax
from jax.experimental import pallas as pl
from jax.experimental.pallas import tpu as pltpu
from jax.experimental.pallas import tpu_sc as plsc

_H = 16
_S = 2048
_MAXD = 128
_DLEN = 4224
_TPAD = 264
_LEAD = 4
_GROUPS = _S // 2 // 8


def _bias_body(table_hbm, out_hbm, tab_v, d8_v, sem):
    core = lax.axis_index("c")
    sub = lax.axis_index("s")
    wid = sub * 2 + core
    h = wid // 2
    half = wid % 2
    i_base = half * (_S // 2)

    pltpu.sync_copy(table_hbm, tab_v)

    zeros = jnp.zeros((16,), jnp.int32)
    hvec = zeros + h
    t0 = plsc.load_gather(tab_v, [hvec, zeros])
    t256 = plsc.load_gather(tab_v, [hvec, zeros + 2 * _MAXD])

    def fill_left(c, _):
        d8_v[7, pl.ds(c * 16, 16)] = t0
        return _

    lax.fori_loop(0, 120, fill_left, None)

    def fill_mid(c, _):
        d8_v[7, pl.ds(1920 + c * 16, 16)] = tab_v[h, pl.ds(1 + c * 16, 16)]
        return _

    lax.fori_loop(0, 16, fill_mid, None)

    def fill_right(c, _):
        d8_v[7, pl.ds(2176 + c * 16, 16)] = t256
        return _

    lax.fori_loop(0, 128, fill_right, None)

    for t in range(7):
        s = 7 - t

        def shift(c, _, s=s, t=t):
            d8_v[t, pl.ds(c * 16, 16)] = d8_v[7, pl.ds(c * 16 + s, 16)]
            return _

        lax.fori_loop(0, 256, shift, None)

    rg_base = half * (_S // 2 // 8)

    def group(g, _):
        i0 = i_base + g * 8
        rg = rg_base + g
        start8 = 2040 - i0
        for ct in range(16):
            pltpu.make_async_copy(
                d8_v.at[:, pl.ds(start8 + ct * 128, 128)],
                out_hbm.at[h, rg, ct],
                sem,
            ).start()

        @pl.when(g >= _LEAD)
        def _drain():
            for _i in range(16):
                pltpu.make_async_copy(
                    d8_v.at[:, pl.ds(0, 128)],
                    out_hbm.at[h, rg_base, 0],
                    sem,
                ).wait()

        return _

    lax.fori_loop(0, _GROUPS, group, None)
    for _ in range(_LEAD * 16):
        pltpu.make_async_copy(
            d8_v.at[:, pl.ds(0, 128)],
            out_hbm.at[0, 0, 0],
            sem,
        ).wait()


@functools.partial(jax.jit, static_argnums=())
def _bias_call(tableT):
    mesh = plsc.VectorSubcoreMesh(core_axis_name="c", subcore_axis_name="s")
    return pl.kernel(
        _bias_body,
        out_type=jax.ShapeDtypeStruct((_H, _S // 8, _S // 128, 8, 128), jnp.float32),
        mesh=mesh,
        scratch_types=[
            pltpu.VMEM((_H, _TPAD), jnp.float32),
            pltpu.VMEM((8, _DLEN), jnp.float32),
            pltpu.SemaphoreType.DMA,
        ],
        compiler_params=pltpu.CompilerParams(
            use_tc_tiling_on_sc=False, needs_layout_passes=False
        ),
    )(tableT)


def kernel(seq_len, relative_bias):
    del seq_len
    tableT = jnp.transpose(relative_bias)
    tableT = jnp.pad(tableT, ((0, 0), (0, _TPAD - 2 * _MAXD - 1)))
    out5 = _bias_call(tableT)
    return out5.transpose(0, 1, 3, 2, 4).reshape(_H, _S, _S)


# speedup vs baseline: 150.6970x; 1.0564x over previous
import functools

import jax
import jax.numpy as jnp
from jax import lax
from jax.experimental import pallas as pl
from jax.experimental.pallas import tpu as pltpu
from jax.experimental.pallas import tpu_sc as plsc

_H = 16
_S = 2048
_MAXD = 128
_DLEN = 4224
_LEAD = 4
_GROUPS = _S // 2 // 8


def _bias_body(table_hbm, out_hbm, tab_v, d8_v, sem):
    core = lax.axis_index("c")
    sub = lax.axis_index("s")
    wid = sub * 2 + core
    h = wid // 2
    half = wid % 2
    i_base = half * (_S // 2)

    pltpu.sync_copy(table_hbm, tab_v)

    zeros = jnp.zeros((16,), jnp.int32)
    iota = lax.iota(jnp.int32, 16)
    hvec = zeros + h
    t0 = plsc.load_gather(tab_v, [zeros, hvec])
    t256 = plsc.load_gather(tab_v, [zeros + 2 * _MAXD, hvec])

    def fill_left(c, carry):
        d8_v[7, pl.ds(c * 16, 16)] = t0
        return carry

    lax.fori_loop(0, 120, fill_left, None, unroll=8)

    for c in range(16):
        d8_v[7, pl.ds(1920 + c * 16, 16)] = plsc.load_gather(
            tab_v, [iota + (1 + c * 16), hvec]
        )

    def fill_right(c, carry):
        d8_v[7, pl.ds(2176 + c * 16, 16)] = t256
        return carry

    lax.fori_loop(0, 128, fill_right, None, unroll=8)

    mbase = (1 - half) * 1024
    for t in range(7):
        s = 7 - t

        def shift(c, carry, s=s, t=t):
            o = mbase + c * 16
            d8_v[t, pl.ds(o, 16)] = d8_v[7, pl.ds(o + s, 16)]
            return carry

        lax.fori_loop(0, 192, shift, None, unroll=8)

    rg_base = half * (_S // 2 // 8)

    def group(g, carry):
        i0 = i_base + g * 8
        rg = rg_base + g
        start8 = 2040 - i0
        for ct in range(16):
            pltpu.make_async_copy(
                d8_v.at[:, pl.ds(start8 + ct * 128, 128)],
                out_hbm.at[h, rg, ct],
                sem,
            ).start()

        @pl.when(g >= _LEAD)
        def _drain():
            for _i in range(16):
                pltpu.make_async_copy(
                    d8_v.at[:, pl.ds(0, 128)],
                    out_hbm.at[h, rg_base, 0],
                    sem,
                ).wait()

        return carry

    lax.fori_loop(0, _GROUPS, group, None)
    for _ in range(_LEAD * 16):
        pltpu.make_async_copy(
            d8_v.at[:, pl.ds(0, 128)],
            out_hbm.at[0, 0, 0],
            sem,
        ).wait()


@jax.jit
def _bias_call(table):
    mesh = plsc.VectorSubcoreMesh(core_axis_name="c", subcore_axis_name="s")
    return pl.kernel(
        _bias_body,
        out_type=jax.ShapeDtypeStruct((_H, _S // 8, _S // 128, 8, 128), jnp.float32),
        mesh=mesh,
        scratch_types=[
            pltpu.VMEM((2 * _MAXD + 1, _H), jnp.float32),
            pltpu.VMEM((8, _DLEN), jnp.float32),
            pltpu.SemaphoreType.DMA,
        ],
        compiler_params=pltpu.CompilerParams(
            use_tc_tiling_on_sc=False, needs_layout_passes=False
        ),
    )(table)


def kernel(seq_len, relative_bias):
    del seq_len
    out5 = _bias_call(relative_bias)
    return out5.transpose(0, 1, 3, 2, 4).reshape(_H, _S, _S)


# speedup vs baseline: 152.4784x; 1.0118x over previous
import functools

import jax
import jax.numpy as jnp
from jax import lax
from jax.experimental import pallas as pl
from jax.experimental.pallas import tpu as pltpu
from jax.experimental.pallas import tpu_sc as plsc

_H = 16
_S = 2048
_MAXD = 128
_DLEN = 4224
_LEAD = 4
_GROUPS = _S // 2 // 8


def _bias_body(table_hbm, out_hbm, tab_v, d8_v, dummy_v, sem):
    core = lax.axis_index("c")
    sub = lax.axis_index("s")
    wid = sub * 2 + core
    h = wid // 2
    half = wid % 2
    i_base = half * (_S // 2)

    pltpu.sync_copy(table_hbm, tab_v)

    zeros = jnp.zeros((16,), jnp.int32)
    iota = lax.iota(jnp.int32, 16)
    hvec = zeros + h
    t0 = plsc.load_gather(tab_v, [zeros, hvec])
    t256 = plsc.load_gather(tab_v, [zeros + 2 * _MAXD, hvec])

    def fill_left(c, carry):
        d8_v[7, pl.ds(c * 16, 16)] = t0
        return carry

    lax.fori_loop(0, 120, fill_left, None, unroll=8)

    for c in range(16):
        d8_v[7, pl.ds(1920 + c * 16, 16)] = plsc.load_gather(
            tab_v, [iota + (1 + c * 16), hvec]
        )

    def fill_right(c, carry):
        d8_v[7, pl.ds(2176 + c * 16, 16)] = t256
        return carry

    lax.fori_loop(0, 128, fill_right, None, unroll=8)

    mbase = (1 - half) * 1024
    for t in range(7):
        s = 7 - t

        def shift(c, carry, s=s, t=t):
            o = mbase + c * 16
            d8_v[t, pl.ds(o, 16)] = d8_v[7, pl.ds(o + s, 16)]
            return carry

        lax.fori_loop(0, 192, shift, None, unroll=8)

    rg_base = half * (_S // 2 // 8)

    def group(g, carry):
        i0 = i_base + g * 8
        rg = rg_base + g
        start8 = 2040 - i0
        for ct in range(16):
            pltpu.make_async_copy(
                d8_v.at[:, pl.ds(start8 + ct * 128, 128)],
                out_hbm.at[h, rg, ct],
                sem,
            ).start()

        @pl.when(g >= _LEAD)
        def _drain():
            pltpu.make_async_copy(
                out_hbm.at[h, rg_base],
                dummy_v,
                sem,
            ).wait()

        return carry

    lax.fori_loop(0, _GROUPS, group, None)
    for _ in range(_LEAD):
        pltpu.make_async_copy(
            out_hbm.at[0, 0],
            dummy_v,
            sem,
        ).wait()


@jax.jit
def _bias_call(table):
    mesh = plsc.VectorSubcoreMesh(core_axis_name="c", subcore_axis_name="s")
    return pl.kernel(
        _bias_body,
        out_type=jax.ShapeDtypeStruct((_H, _S // 8, _S // 128, 8, 128), jnp.float32),
        mesh=mesh,
        scratch_types=[
            pltpu.VMEM((2 * _MAXD + 1, _H), jnp.float32),
            pltpu.VMEM((8, _DLEN), jnp.float32),
            pltpu.VMEM((_S // 128, 8, 128), jnp.float32),
            pltpu.SemaphoreType.DMA,
        ],
        compiler_params=pltpu.CompilerParams(
            use_tc_tiling_on_sc=False, needs_layout_passes=False
        ),
    )(table)


def kernel(seq_len, relative_bias):
    del seq_len
    out5 = _bias_call(relative_bias)
    return out5.transpose(0, 1, 3, 2, 4).reshape(_H, _S, _S)
